# Initial kernel scaffold; baseline (speedup 1.0000x reference)
import jax
import jax.numpy as jnp
from jax.experimental import pallas as pl


def kernel(phrase, W, pos_emb):
    raise NotImplementedError("write your pallas kernel here")


# speedup vs baseline: 1.7427x; 1.7427x over previous
import functools

import jax
import jax.numpy as jnp
from jax import lax
from jax.experimental import pallas as pl
from jax.experimental.pallas import tpu as pltpu
from jax.experimental.pallas import tpu_sc as plsc

VOCAB = 1000000
HID = 64
B = 16384
L = 50
POS_ROWS = 128

NC = 2
NS = 16
NW = NC * NS

N = B * L
ROWS_PW = N // NW
CHUNK = 800
NCHUNK = ROWS_PW // CHUNK
IDX_MINOR = 100
DMAS_PER_CHUNK = CHUNK // IDX_MINOR
PHRASES_PER_CHUNK = CHUNK // L
VECS = HID // 16
POS_COPY = 56


def _sc_body(idx_hbm, w_hbm, pos_hbm, out_hbm, idx_v, buf, pos_v, gsem):
    wid = lax.axis_index("s") * NC + lax.axis_index("c")
    pltpu.sync_copy(pos_hbm.at[pl.ds(0, POS_COPY)], pos_v)

    def chunk_body(c, carry):
        row0 = pl.multiple_of(wid * ROWS_PW + c * CHUNK, CHUNK)
        irow = pl.multiple_of(wid * (ROWS_PW // IDX_MINOR) + c * DMAS_PER_CHUNK,
                              DMAS_PER_CHUNK)
        pltpu.sync_copy(idx_hbm.at[pl.ds(irow, DMAS_PER_CHUNK)], idx_v)
        copies = []
        for j in range(DMAS_PER_CHUNK):
            copies.append(pltpu.async_copy(
                w_hbm.at[idx_v.at[j]],
                buf.at[pl.ds(j * IDX_MINOR, IDX_MINOR)], gsem))
        for cp in copies:
            cp.wait()

        def pos_body(p, carry2):
            for q in range(VECS):
                pv = pos_v[p, pl.ds(q * 16, 16)]
                for r in range(PHRASES_PER_CHUNK):
                    row = r * L + p
                    buf[row, pl.ds(q * 16, 16)] = buf[row, pl.ds(q * 16, 16)] + pv
            return carry2

        lax.fori_loop(0, L, pos_body, 0)
        pltpu.sync_copy(buf, out_hbm.at[pl.ds(row0, CHUNK)])
        return carry

    lax.fori_loop(0, NCHUNK, chunk_body, 0)


@jax.jit
def _phrase_embedding_sc(idx2d, w, pos):
    mesh = plsc.VectorSubcoreMesh(
        core_axis_name="c", subcore_axis_name="s",
        num_cores=NC, num_subcores=NS)
    call = functools.partial(
        pl.kernel,
        out_type=jax.ShapeDtypeStruct((N, HID), jnp.float32),
        mesh=mesh,
        scratch_types=[
            pltpu.VMEM((DMAS_PER_CHUNK, IDX_MINOR), jnp.int32),
            pltpu.VMEM((CHUNK, HID), jnp.float32),
            pltpu.VMEM((POS_COPY, HID), jnp.float32),
            pltpu.SemaphoreType.DMA,
        ],
        compiler_params=pltpu.CompilerParams(use_tc_tiling_on_sc=False),
    )(_sc_body)
    return call(idx2d, w, pos)


def kernel(phrase, W, pos_emb):
    idx2d = phrase.astype(jnp.int32).reshape(N // IDX_MINOR, IDX_MINOR)
    out = _phrase_embedding_sc(idx2d, W, pos_emb)
    return out.reshape(B, L, HID)


# speedup vs baseline: 1.8652x; 1.0703x over previous
import functools

import jax
import jax.numpy as jnp
from jax import lax
from jax.experimental import pallas as pl
from jax.experimental.pallas import tpu as pltpu
from jax.experimental.pallas import tpu_sc as plsc

VOCAB = 1000000
HID = 64
B = 16384
L = 50
POS_ROWS = 128

NC = 2
NS = 16
NW = NC * NS

N = B * L
ROWS_PW = N // NW
CHUNK = 800
NCHUNK = ROWS_PW // CHUNK
IDX_MINOR = 100
DMAS_PER_CHUNK = CHUNK // IDX_MINOR
PHRASES_PER_CHUNK = CHUNK // L
VECS = HID // 16
POS_COPY = 56


def _sc_body(idx_hbm, w_hbm, pos_hbm, out_hbm, idx_v, buf, pos_v,
             gsem0, gsem1, osem0, osem1):
    wid = lax.axis_index("s") * NC + lax.axis_index("c")
    pltpu.sync_copy(pos_hbm.at[pl.ds(0, POS_COPY)], pos_v)
    gsems = (gsem0, gsem1)
    osems = (osem0, osem1)

    def load_idx(c, s):
        irow = pl.multiple_of(
            wid * (ROWS_PW // IDX_MINOR) + c * DMAS_PER_CHUNK, DMAS_PER_CHUNK)
        pltpu.sync_copy(idx_hbm.at[pl.ds(irow, DMAS_PER_CHUNK)], idx_v.at[s])

    def gather_copies(s):
        return [pltpu.make_async_copy(
                    w_hbm.at[idx_v.at[s].at[j]],
                    buf.at[s].at[pl.ds(j * IDX_MINOR, IDX_MINOR)], gsems[s])
                for j in range(DMAS_PER_CHUNK)]

    def fire_gather(s):
        for cp in gather_copies(s):
            cp.start()

    def wait_gather(s):
        for cp in gather_copies(s):
            cp.wait()

    def fire_store(c, s):
        row0 = pl.multiple_of(wid * ROWS_PW + c * CHUNK, CHUNK)
        pltpu.async_copy(buf.at[s], out_hbm.at[pl.ds(row0, CHUNK)], osems[s])

    def wait_store(s):
        pltpu.make_async_copy(
            buf.at[s], out_hbm.at[pl.ds(0, CHUNK)], osems[s]).wait()

    def pos_add(s):
        def pos_body(p, carry):
            for q in range(VECS):
                pv = pos_v[p, pl.ds(q * 16, 16)]
                for r in range(PHRASES_PER_CHUNK):
                    row = r * L + p
                    buf[s, row, pl.ds(q * 16, 16)] = (
                        buf[s, row, pl.ds(q * 16, 16)] + pv)
            return carry
        lax.fori_loop(0, L, pos_body, 0)

    load_idx(0, 0)
    fire_gather(0)

    @pl.loop(0, NCHUNK, step=2)
    def _chunks(c0):
        for b in range(2):
            c = c0 + b
            nxt = c + 1

            @pl.when(nxt < NCHUNK)
            def _prefetch():
                load_idx(nxt, 1 - b)

                @pl.when(c >= 1)
                def _drain_prev_store():
                    wait_store(1 - b)

                fire_gather(1 - b)

            wait_gather(b)
            pos_add(b)
            fire_store(c, b)

    wait_store(0)
    wait_store(1)


@jax.jit
def _phrase_embedding_sc(idx2d, w, pos):
    mesh = plsc.VectorSubcoreMesh(
        core_axis_name="c", subcore_axis_name="s",
        num_cores=NC, num_subcores=NS)
    call = functools.partial(
        pl.kernel,
        out_type=jax.ShapeDtypeStruct((N, HID), jnp.float32),
        mesh=mesh,
        scratch_types=[
            pltpu.VMEM((2, DMAS_PER_CHUNK, IDX_MINOR), jnp.int32),
            pltpu.VMEM((2, CHUNK, HID), jnp.float32),
            pltpu.VMEM((POS_COPY, HID), jnp.float32),
            pltpu.SemaphoreType.DMA,
            pltpu.SemaphoreType.DMA,
            pltpu.SemaphoreType.DMA,
            pltpu.SemaphoreType.DMA,
        ],
        compiler_params=pltpu.CompilerParams(use_tc_tiling_on_sc=False),
    )(_sc_body)
    return call(idx2d, w, pos)


def kernel(phrase, W, pos_emb):
    idx2d = phrase.astype(jnp.int32).reshape(N // IDX_MINOR, IDX_MINOR)
    out = _phrase_embedding_sc(idx2d, W, pos_emb)
    return out.reshape(B, L, HID)
